# baseline (speedup 1.0000x reference)
import jax
import jax.numpy as jnp
from jax.experimental import pallas as pl
from jax.experimental.pallas import tpu as pltpu

BS = 8
NP = 1024
NKP = 8
NK1 = NKP + 1
XD = 32
ZD = 64
HD = 64
N = BS * NK1
E = 576
EPAD = E + 8
TP = 32
f32 = jnp.float32


def _pre_body(x2_ref, z_ref, pos_ref, src_ref, dst_ref,
              lw1x_ref, lw1z_ref, lw1p_ref, lb1_ref,
              u_ref, c_ref):
    u_ref[...] = jnp.dot(x2_ref[...], lw1x_ref[...])
    src = src_ref[...]
    dst = dst_ref[...]
    iota_n = jax.lax.broadcasted_iota(jnp.int32, (E, N), 1)
    oh_src = (iota_n == src).astype(f32)
    oh_dst = (iota_n == dst).astype(f32)
    zl = jnp.dot(z_ref[...], lw1z_ref[...])
    pr = jnp.dot(pos_ref[...], lw1p_ref[...])
    c_ref[...] = (jnp.dot(oh_src, zl + pr) - jnp.dot(oh_dst, pr)
                  + lb1_ref[...])


def _main_body(lastpos_ref, u3_ref, src_ref, dst_ref, c_ref,
               lw2_ref, lb2_ref, lw3_ref, lb3_ref,
               gw1_ref, gb1_ref, gw2_ref, gb2_ref, gw3_ref, gb3_ref,
               out_ref, hscr_ref, aggscr_ref):
    src = src_ref[...]
    dst = dst_ref[...]
    b_idx = src // NK1
    u3 = u3_ref[...]

    ue = jnp.broadcast_to(u3[BS - 1][None], (E, TP, HD))
    for b in range(BS - 2, -1, -1):
        m = (b_idx == b)[:, :, None]
        ue = jnp.where(m, u3[b][None], ue)

    h = jax.nn.relu(ue + c_ref[...][:, None, :])
    h2 = h.reshape(E * TP, HD)
    h2 = jax.nn.relu(jnp.dot(h2, lw2_ref[...]) + lb2_ref[...])
    h2 = jax.nn.relu(jnp.dot(h2, lw3_ref[...]) + lb3_ref[...])
    h = h2.reshape(E, TP, HD)

    scanned = h
    k = 1
    while k < E:
        same = jnp.concatenate(
            [jnp.zeros((k, 1), f32), (dst[k:] == dst[:-k]).astype(f32)],
            axis=0)
        shifted = jnp.concatenate(
            [jnp.zeros((k, TP, HD), f32), scanned[:-k]], axis=0)
        scanned = jnp.maximum(scanned, shifted * same[:, :, None])
        k *= 2

    hscr_ref[:E] = scanned
    hscr_ref[E:] = jnp.zeros((EPAD - E, TP, HD), f32)
    for n in range(N):
        row = lastpos_ref[0, n]
        aggscr_ref[n:n + 1] = hscr_ref[pl.ds(row, 1)]

    agg = aggscr_ref[...].reshape(N * TP, HD)
    g = jax.nn.relu(jnp.dot(agg, gw1_ref[...]) + gb1_ref[...])
    g = jax.nn.relu(jnp.dot(g, gw2_ref[...]) + gb2_ref[...])
    o = jnp.dot(g, gw3_ref[...]) + gb3_ref[...]
    out_ref[...] = o.reshape(N, TP, 3)


@jax.jit
def kernel(x, z, pos, edge_index, lw1, lb1, lw2, lb2, lw3, lb3,
           gw1, gb1, gw2, gb2, gw3, gb3):
    zflat = z.reshape(N, ZD)
    pos72 = pos.reshape(N, 3)

    src = edge_index[0].astype(jnp.int32)
    dst = edge_index[1].astype(jnp.int32)
    order = jnp.argsort(dst)
    src_s = src[order].reshape(E, 1)
    dst_s = dst[order].reshape(E, 1)
    lastpos = jnp.full((N,), -1, jnp.int32).at[dst_s[:, 0]].max(
        jnp.arange(E, dtype=jnp.int32))
    lastpos = jnp.where(lastpos < 0, E, lastpos).reshape(1, N)

    lw1x = lw1[:XD]
    lw1z = lw1[XD:XD + ZD]
    lw1p = lw1[XD + ZD:]

    full = lambda s: pl.BlockSpec(s, lambda *a: tuple(0 for _ in s))

    u, c = pl.pallas_call(
        _pre_body,
        in_specs=[full((BS * NP, XD)), full((N, ZD)), full((N, 3)),
                  full((E, 1)), full((E, 1)),
                  full((XD, HD)), full((ZD, HD)), full((3, HD)),
                  full((1, HD))],
        out_specs=[full((BS * NP, HD)), full((E, HD))],
        out_shape=[jax.ShapeDtypeStruct((BS * NP, HD), f32),
                   jax.ShapeDtypeStruct((E, HD), f32)],
    )(x.reshape(BS * NP, XD), zflat, pos72, src_s, dst_s,
      lw1x, lw1z, lw1p, lb1.reshape(1, HD))

    u3 = u.reshape(BS, NP, HD)

    grid = NP // TP
    out = pl.pallas_call(
        _main_body,
        grid=(grid,),
        in_specs=[
            pl.BlockSpec(memory_space=pltpu.SMEM),
            pl.BlockSpec((BS, TP, HD), lambda i: (0, i, 0)),
            full((E, 1)),
            full((E, 1)),
            full((E, HD)),
            full((HD, HD)), full((1, HD)),
            full((HD, HD)), full((1, HD)),
            full((HD, HD)), full((1, HD)),
            full((HD, HD)), full((1, HD)),
            full((HD, 3)), full((1, 3)),
        ],
        out_specs=pl.BlockSpec((N, TP, 3), lambda i: (0, i, 0)),
        out_shape=jax.ShapeDtypeStruct((N, NP, 3), f32),
        scratch_shapes=[pltpu.VMEM((EPAD, TP, HD), f32),
                        pltpu.VMEM((N, TP, HD), f32)],
    )(lastpos, u3, src_s, dst_s, c,
      lw2, lb2.reshape(1, HD), lw3, lb3.reshape(1, HD),
      gw1, gb1.reshape(1, HD), gw2, gb2.reshape(1, HD),
      gw3, gb3.reshape(1, 3))

    final = out.reshape(BS, NK1, NP, 3).transpose(0, 2, 1, 3)
    return final[:, :, :NKP, :], final[:, :, NKP:, :]


# speedup vs baseline: 1.4471x; 1.4471x over previous
import jax
import jax.numpy as jnp
from jax.experimental import pallas as pl
from jax.experimental.pallas import tpu as pltpu

BS = 8
NP = 1024
NKP = 8
NK1 = NKP + 1
XD = 32
ZD = 64
HD = 64
N = BS * NK1
E = 576
EPAD = E + 8
TP = 64
f32 = jnp.float32
bf16 = jnp.bfloat16


def _pre_body(x2_ref, z_ref, pos_ref, src_ref, dst_ref,
              lw1x_ref, lw1z_ref, lw1p_ref, lb1_ref,
              u_ref, c_ref):
    u_ref[...] = jnp.dot(x2_ref[...], lw1x_ref[...])
    src = src_ref[...]
    dst = dst_ref[...]
    iota_n = jax.lax.broadcasted_iota(jnp.int32, (E, N), 1)
    oh_src = (iota_n == src).astype(f32)
    oh_dst = (iota_n == dst).astype(f32)
    zl = jnp.dot(z_ref[...], lw1z_ref[...])
    pr = jnp.dot(pos_ref[...], lw1p_ref[...])
    c_ref[...] = (jnp.dot(oh_src, zl + pr) - jnp.dot(oh_dst, pr)
                  + lb1_ref[...])


def _main_body(lastpos_ref, u3_ref, src_ref, dst_ref, c_ref,
               lw2_ref, lb2_ref, lw3_ref, lb3_ref,
               gw1_ref, gb1_ref, gw2_ref, gb2_ref, gw3_ref, gb3_ref,
               out_ref, hscr_ref, aggscr_ref):
    src = src_ref[...]
    dst = dst_ref[...]
    b_idx = src // NK1
    u3 = u3_ref[...].astype(bf16)

    ue = jnp.broadcast_to(u3[BS - 1][None], (E, TP, HD))
    for b in range(BS - 2, -1, -1):
        m = (b_idx == b)[:, :, None]
        ue = jnp.where(m, u3[b][None], ue)

    h = jax.nn.relu(ue + c_ref[...].astype(bf16)[:, None, :])
    h2 = h.reshape(E * TP, HD)
    h2 = jax.nn.relu(jnp.dot(h2, lw2_ref[...].astype(bf16),
                             preferred_element_type=f32)
                     + lb2_ref[...]).astype(bf16)
    h2 = jax.nn.relu(jnp.dot(h2, lw3_ref[...].astype(bf16),
                             preferred_element_type=f32)
                     + lb3_ref[...]).astype(bf16)
    h = h2.reshape(E, TP, HD)

    scanned = h
    k = 1
    while k < E:
        same = jnp.concatenate(
            [jnp.zeros((k, 1), bf16), (dst[k:] == dst[:-k]).astype(bf16)],
            axis=0)
        shifted = jnp.concatenate(
            [jnp.zeros((k, TP, HD), bf16), scanned[:-k]], axis=0)
        scanned = jnp.maximum(scanned, shifted * same[:, :, None])
        k *= 2

    hscr_ref[:E] = scanned
    hscr_ref[E:] = jnp.zeros((EPAD - E, TP, HD), bf16)
    for n in range(N):
        row = lastpos_ref[0, n]
        aggscr_ref[n:n + 1] = hscr_ref[pl.ds(row, 1)]

    agg = aggscr_ref[...].reshape(N * TP, HD)
    g = jax.nn.relu(jnp.dot(agg, gw1_ref[...].astype(bf16),
                            preferred_element_type=f32)
                    + gb1_ref[...]).astype(bf16)
    g = jax.nn.relu(jnp.dot(g, gw2_ref[...].astype(bf16),
                            preferred_element_type=f32)
                    + gb2_ref[...]).astype(bf16)
    o = jnp.dot(g, gw3_ref[...].astype(bf16),
                preferred_element_type=f32) + gb3_ref[...]
    out_ref[...] = o.reshape(N, TP, 3)


@jax.jit
def kernel(x, z, pos, edge_index, lw1, lb1, lw2, lb2, lw3, lb3,
           gw1, gb1, gw2, gb2, gw3, gb3):
    zflat = z.reshape(N, ZD)
    pos72 = pos.reshape(N, 3)

    src = edge_index[0].astype(jnp.int32)
    dst = edge_index[1].astype(jnp.int32)
    order = jnp.argsort(dst)
    src_s = src[order].reshape(E, 1)
    dst_s = dst[order].reshape(E, 1)
    lastpos = jnp.full((N,), -1, jnp.int32).at[dst_s[:, 0]].max(
        jnp.arange(E, dtype=jnp.int32))
    lastpos = jnp.where(lastpos < 0, E, lastpos).reshape(1, N)

    lw1x = lw1[:XD]
    lw1z = lw1[XD:XD + ZD]
    lw1p = lw1[XD + ZD:]

    full = lambda s: pl.BlockSpec(s, lambda *a: tuple(0 for _ in s))

    u, c = pl.pallas_call(
        _pre_body,
        in_specs=[full((BS * NP, XD)), full((N, ZD)), full((N, 3)),
                  full((E, 1)), full((E, 1)),
                  full((XD, HD)), full((ZD, HD)), full((3, HD)),
                  full((1, HD))],
        out_specs=[full((BS * NP, HD)), full((E, HD))],
        out_shape=[jax.ShapeDtypeStruct((BS * NP, HD), f32),
                   jax.ShapeDtypeStruct((E, HD), f32)],
    )(x.reshape(BS * NP, XD), zflat, pos72, src_s, dst_s,
      lw1x, lw1z, lw1p, lb1.reshape(1, HD))

    u3 = u.reshape(BS, NP, HD)

    grid = NP // TP
    out = pl.pallas_call(
        _main_body,
        grid=(grid,),
        in_specs=[
            pl.BlockSpec(memory_space=pltpu.SMEM),
            pl.BlockSpec((BS, TP, HD), lambda i: (0, i, 0)),
            full((E, 1)),
            full((E, 1)),
            full((E, HD)),
            full((HD, HD)), full((1, HD)),
            full((HD, HD)), full((1, HD)),
            full((HD, HD)), full((1, HD)),
            full((HD, HD)), full((1, HD)),
            full((HD, 3)), full((1, 3)),
        ],
        out_specs=pl.BlockSpec((N, TP, 3), lambda i: (0, i, 0)),
        out_shape=jax.ShapeDtypeStruct((N, NP, 3), f32),
        scratch_shapes=[pltpu.VMEM((EPAD, TP, HD), bf16),
                        pltpu.VMEM((N, TP, HD), bf16)],
    )(lastpos, u3, src_s, dst_s, c,
      lw2, lb2.reshape(1, HD), lw3, lb3.reshape(1, HD),
      gw1, gb1.reshape(1, HD), gw2, gb2.reshape(1, HD),
      gw3, gb3.reshape(1, 3))

    final = out.reshape(BS, NK1, NP, 3).transpose(0, 2, 1, 3)
    return final[:, :, :NKP, :], final[:, :, NKP:, :]


# speedup vs baseline: 1.9617x; 1.3556x over previous
import jax
import jax.numpy as jnp
from jax.experimental import pallas as pl
from jax.experimental.pallas import tpu as pltpu

BS = 8
NP = 1024
NKP = 8
NK1 = NKP + 1
XD = 32
ZD = 64
HD = 64
N = BS * NK1
E = 576
EPAD = E + 8
TP = 64
G = 8
NG = E // G
f32 = jnp.float32
bf16 = jnp.bfloat16
i32 = jnp.int32


def _pre_body(x2_ref, z_ref, pos_ref, src_ref, dst_ref, dstT_ref,
              lw1x_ref, lw1z_ref, lw1p_ref, lb1_ref,
              u_ref, c_ref, b_ref, dsts_ref, lastpos_ref):
    u_ref[...] = jnp.dot(x2_ref[...], lw1x_ref[...])

    src = src_ref[...]
    dst = dst_ref[...]
    dstT = dstT_ref[...]

    iota_n = jax.lax.broadcasted_iota(i32, (E, N), 1)
    oh_src = (iota_n == src).astype(f32)
    oh_dst = (iota_n == dst).astype(f32)
    zl = jnp.dot(z_ref[...], lw1z_ref[...])
    pr = jnp.dot(pos_ref[...], lw1p_ref[...])
    c = (jnp.dot(oh_src, zl + pr) - jnp.dot(oh_dst, pr)
         + lb1_ref[...])

    iota_e_row = jax.lax.broadcasted_iota(i32, (E, E), 0)
    iota_e_col = jax.lax.broadcasted_iota(i32, (E, E), 1)
    lt = (dst < dstT).astype(f32)
    eq_tri = ((dst == dstT) & (iota_e_row <= iota_e_col)).astype(f32)
    rankT = jnp.dot(jnp.ones((1, E), f32), lt + eq_tri) - 1.0

    rankT_i = (rankT + 0.5).astype(i32)
    perm = (jax.lax.broadcasted_iota(i32, (E, E), 0) == rankT_i).astype(f32)
    vals = jnp.concatenate(
        [dst.astype(f32), (src // NK1).astype(f32)], axis=1)
    sorted_vals = jnp.dot(perm, vals)
    dsts_ref[...] = sorted_vals[:, 0:1].astype(i32)
    b_ref[...] = sorted_vals[:, 1:2].astype(i32)
    c_ref[...] = jnp.dot(perm, c)

    le = (dst <= jax.lax.broadcasted_iota(i32, (E, N), 1)).astype(f32)
    cnt = jnp.dot(jnp.ones((1, E), f32), le)
    prev = jnp.concatenate([jnp.zeros((1, 1), f32), cnt[:, :-1]], axis=1)
    lastpos_ref[...] = jnp.where(cnt > prev, cnt - 1.0,
                                 float(E)).astype(i32)


def _seg_scan(vals, seg, steps):
    rows = vals.shape[0]
    k = 1
    for _ in range(steps):
        same = jnp.concatenate(
            [jnp.zeros((k, 1), bf16), (seg[k:] == seg[:-k]).astype(bf16)],
            axis=0)
        shifted = jnp.concatenate(
            [jnp.zeros((k,) + vals.shape[1:], bf16), vals[:-k]], axis=0)
        vals = jnp.maximum(vals, shifted * same[:, :, None])
        k *= 2
        if k >= rows:
            break
    return vals


def _main_body(lastpos_ref, u3_ref, b_ref, dst_ref, c_ref,
               lw2_ref, lb2_ref, lw3_ref, lb3_ref,
               gw1_ref, gb1_ref, gw2_ref, gb2_ref, gw3_ref, gb3_ref,
               out_ref, hscr_ref, aggscr_ref):
    dst = dst_ref[...]
    b_idx = b_ref[...]
    u3 = u3_ref[...].astype(bf16)

    ue = jnp.broadcast_to(u3[BS - 1][None], (E, TP, HD))
    for b in range(BS - 2, -1, -1):
        m = (b_idx == b)[:, :, None]
        ue = jnp.where(m, u3[b][None], ue)

    h = jax.nn.relu(ue + c_ref[...].astype(bf16)[:, None, :])
    h2 = h.reshape(E * TP, HD)
    h2 = jax.nn.relu(jnp.dot(h2, lw2_ref[...].astype(bf16),
                             preferred_element_type=f32)
                     + lb2_ref[...]).astype(bf16)
    h2 = jax.nn.relu(jnp.dot(h2, lw3_ref[...].astype(bf16),
                             preferred_element_type=f32)
                     + lb3_ref[...]).astype(bf16)
    h = h2.reshape(E, TP, HD)

    a = _seg_scan(h, dst, 3)
    a4 = a.reshape(NG, G, TP, HD)
    dst4 = dst.reshape(NG, G, 1)
    tails = a4[:, G - 1]
    gdst = dst4[:, G - 1]
    bs = _seg_scan(tails, gdst, 7)
    bprev = jnp.concatenate(
        [jnp.zeros((1, TP, HD), bf16), bs[:-1]], axis=0)
    gdprev = jnp.concatenate(
        [jnp.full((1, 1), -1, i32), gdst[:-1]], axis=0)
    m = (dst4 == gdprev[:, None, :]).astype(bf16)
    contrib = bprev[:, None, :, :] * m[:, :, :, None]
    scanned = jnp.maximum(a4, contrib).reshape(E, TP, HD)

    hscr_ref[:E] = scanned
    hscr_ref[E:] = jnp.zeros((EPAD - E, TP, HD), bf16)
    for n in range(N):
        row = lastpos_ref[0, n]
        aggscr_ref[n:n + 1] = hscr_ref[pl.ds(row, 1)]

    agg = aggscr_ref[...].reshape(N * TP, HD)
    g = jax.nn.relu(jnp.dot(agg, gw1_ref[...].astype(bf16),
                            preferred_element_type=f32)
                    + gb1_ref[...]).astype(bf16)
    g = jax.nn.relu(jnp.dot(g, gw2_ref[...].astype(bf16),
                            preferred_element_type=f32)
                    + gb2_ref[...]).astype(bf16)
    o = jnp.dot(g, gw3_ref[...].astype(bf16),
                preferred_element_type=f32) + gb3_ref[...]
    out_ref[...] = o.reshape(N, TP, 3)


@jax.jit
def kernel(x, z, pos, edge_index, lw1, lb1, lw2, lb2, lw3, lb3,
           gw1, gb1, gw2, gb2, gw3, gb3):
    zflat = z.reshape(N, ZD)
    pos72 = pos.reshape(N, 3)

    src = edge_index[0].astype(i32).reshape(E, 1)
    dst = edge_index[1].astype(i32).reshape(E, 1)
    dstT = edge_index[1].astype(i32).reshape(1, E)

    lw1x = lw1[:XD]
    lw1z = lw1[XD:XD + ZD]
    lw1p = lw1[XD + ZD:]

    full = lambda s: pl.BlockSpec(s, lambda *a: tuple(0 for _ in s))

    u, c, b_s, dst_s, lastpos = pl.pallas_call(
        _pre_body,
        in_specs=[full((BS * NP, XD)), full((N, ZD)), full((N, 3)),
                  full((E, 1)), full((E, 1)), full((1, E)),
                  full((XD, HD)), full((ZD, HD)), full((3, HD)),
                  full((1, HD))],
        out_specs=[full((BS * NP, HD)), full((E, HD)), full((E, 1)),
                   full((E, 1)), full((1, N))],
        out_shape=[jax.ShapeDtypeStruct((BS * NP, HD), f32),
                   jax.ShapeDtypeStruct((E, HD), f32),
                   jax.ShapeDtypeStruct((E, 1), i32),
                   jax.ShapeDtypeStruct((E, 1), i32),
                   jax.ShapeDtypeStruct((1, N), i32)],
    )(x.reshape(BS * NP, XD), zflat, pos72, src, dst, dstT,
      lw1x, lw1z, lw1p, lb1.reshape(1, HD))

    u3 = u.reshape(BS, NP, HD)

    grid = NP // TP
    out = pl.pallas_call(
        _main_body,
        grid=(grid,),
        in_specs=[
            pl.BlockSpec(memory_space=pltpu.SMEM),
            pl.BlockSpec((BS, TP, HD), lambda i: (0, i, 0)),
            full((E, 1)),
            full((E, 1)),
            full((E, HD)),
            full((HD, HD)), full((1, HD)),
            full((HD, HD)), full((1, HD)),
            full((HD, HD)), full((1, HD)),
            full((HD, HD)), full((1, HD)),
            full((HD, 3)), full((1, 3)),
        ],
        out_specs=pl.BlockSpec((N, TP, 3), lambda i: (0, i, 0)),
        out_shape=jax.ShapeDtypeStruct((N, NP, 3), f32),
        scratch_shapes=[pltpu.VMEM((EPAD, TP, HD), bf16),
                        pltpu.VMEM((N, TP, HD), bf16)],
    )(lastpos, u3, b_s, dst_s, c,
      lw2, lb2.reshape(1, HD), lw3, lb3.reshape(1, HD),
      gw1, gb1.reshape(1, HD), gw2, gb2.reshape(1, HD),
      gw3, gb3.reshape(1, 3))

    final = out.reshape(BS, NK1, NP, 3).transpose(0, 2, 1, 3)
    return final[:, :, :NKP, :], final[:, :, NKP:, :]
